# baseline (speedup 1.0000x reference)
import functools

import jax
import jax.numpy as jnp
from jax import lax
from jax.experimental import pallas as pl
from jax.experimental.pallas import tpu as pltpu
from jax.experimental.pallas import tpu_sc as plsc

N = 10000
D = 128
E = 320000
NC = 2
NS = 16
NW = NC * NS
CH = 64
NSLOT = 4
EPT = 10240
NCH = EPT // CH
NG = NCH // NSLOT
EPAD = NW * EPT
NPAD = 10112
RPT = NPAD // NS

_mesh = plsc.VectorSubcoreMesh(core_axis_name="c", subcore_axis_name="s")


@functools.partial(
    pl.kernel,
    out_type=jax.ShapeDtypeStruct((NW, NPAD), jnp.float32),
    mesh=_mesh,
    scratch_types=[
        pltpu.VMEM((EPT,), jnp.int32),
        pltpu.VMEM((NPAD,), jnp.float32),
    ],
    compiler_params=pltpu.CompilerParams(needs_layout_passes=False),
)
def _deg(eidx_hbm, out_hbm, eidx_v, deg_v):
    c = lax.axis_index("c")
    s = lax.axis_index("s")
    wid = c * NS + s

    def zero(i, _):
        deg_v[pl.ds(i * 16, 16)] = jnp.zeros((16,), jnp.float32)
        return 0

    lax.fori_loop(0, NPAD // 16, zero, 0)
    pltpu.sync_copy(eidx_hbm.at[wid], eidx_v)
    ones = jnp.ones((16,), jnp.float32)

    def body(i, _):
        idx = eidx_v[pl.ds(i * 16, 16)] >> 16
        plsc.addupdate_scatter(deg_v, [idx], ones)
        return 0

    lax.fori_loop(0, EPT // 16, body, 0)
    pltpu.sync_copy(deg_v, out_hbm.at[wid])


@functools.partial(
    pl.kernel,
    out_type=jax.ShapeDtypeStruct((NC, NPAD, D), jnp.float32),
    mesh=_mesh,
    scratch_types=[
        pltpu.VMEM_SHARED((NPAD, D), jnp.float32),
        pltpu.VMEM((EPT,), jnp.int32),
        [pltpu.VMEM((CH,), jnp.int32) for _ in range(NSLOT)],
        [pltpu.VMEM((CH,), jnp.int32) for _ in range(NSLOT)],
        [pltpu.VMEM((CH, D), jnp.float32) for _ in range(NSLOT)],
        [pltpu.SemaphoreType.DMA for _ in range(NSLOT)],
        [pltpu.SemaphoreType.DMA for _ in range(NSLOT)],
    ],
    compiler_params=pltpu.CompilerParams(needs_layout_passes=False),
)
def _edge(y_hbm, zeros_hbm, eidx_hbm, out_hbm, acc, eidx_v, sb, db, rows,
          gsem, ssem):
    c = lax.axis_index("c")
    s = lax.axis_index("s")
    wid = c * NS + s
    rs = s * RPT

    @pl.when(c == 0)
    def _():
        pltpu.sync_copy(y_hbm.at[pl.ds(rs, RPT)], acc.at[pl.ds(rs, RPT)])

    @pl.when(c != 0)
    def _():
        pltpu.sync_copy(zeros_hbm.at[pl.ds(rs, RPT)], acc.at[pl.ds(rs, RPT)])

    pltpu.sync_copy(eidx_hbm.at[wid], eidx_v)
    plsc.subcore_barrier()

    def unpack(j, b):
        for k in range(CH // 16):
            v = eidx_v[pl.ds(j * CH + k * 16, 16)]
            sb[b][pl.ds(k * 16, 16)] = v & 0xFFFF
            db[b][pl.ds(k * 16, 16)] = v >> 16

    def gather(j, b):
        unpack(j, b)
        pltpu.async_copy(y_hbm.at[sb[b]], rows[b], gsem[b])

    def gwait(b):
        pltpu.make_async_copy(y_hbm.at[sb[b]], rows[b], gsem[b]).wait()

    def scatter(b):
        pltpu.async_copy(rows[b], acc.at[db[b]], ssem[b], add=True)

    def swait(b):
        pltpu.make_async_copy(rows[b], acc.at[db[b]], ssem[b]).wait()

    for b in range(NSLOT):
        gather(b, b)

    def body(i, _):
        for b in range(NSLOT):
            gwait(b)
            scatter(b)
        for b in range(NSLOT):
            swait(b)
            gather(NSLOT * (i + 1) + b, b)
        return 0

    lax.fori_loop(0, NG - 1, body, 0)
    for b in range(NSLOT):
        gwait(b)
        scatter(b)
    for b in range(NSLOT):
        swait(b)
    plsc.subcore_barrier()
    pltpu.sync_copy(acc.at[pl.ds(rs, RPT)], out_hbm.at[c].at[pl.ds(rs, RPT)])


def _prep_body(parts_ref, feat_ref, dis_ref, y_ref):
    deg = jnp.sum(parts_ref[...], axis=0) + 1.0
    dis = lax.rsqrt(deg)
    dis_ref[...] = dis
    y_ref[...] = dis[:, None] * feat_ref[...]


def _comb_body(parts_ref, dis_ref, x1_ref, y2_ref):
    s = parts_ref[0] + parts_ref[1]
    dis = dis_ref[...]
    x1f = dis[:, None] * s
    x1_ref[...] = x1f[:N]
    y2_ref[...] = dis[:, None] * x1f


def _final_body(parts_ref, dis_ref, lin_ref, logp_ref, out_ref, x2_ref):
    s = parts_ref[0] + parts_ref[1]
    x2f = dis_ref[...][:, None] * s
    x2 = x2f[:N]
    o = jnp.dot(x2, lin_ref[...], preferred_element_type=jnp.float32)
    m = jnp.max(o, axis=1, keepdims=True)
    lse = m + jnp.log(jnp.sum(jnp.exp(o - m), axis=1, keepdims=True))
    logp_ref[...] = o - lse
    out_ref[...] = o
    x2_ref[...] = x2


_prep = pl.pallas_call(
    _prep_body,
    out_shape=[
        jax.ShapeDtypeStruct((NPAD,), jnp.float32),
        jax.ShapeDtypeStruct((NPAD, D), jnp.float32),
    ],
)

_comb = pl.pallas_call(
    _comb_body,
    out_shape=[
        jax.ShapeDtypeStruct((N, D), jnp.float32),
        jax.ShapeDtypeStruct((NPAD, D), jnp.float32),
    ],
)

_final = pl.pallas_call(
    _final_body,
    out_shape=[
        jax.ShapeDtypeStruct((N, D), jnp.float32),
        jax.ShapeDtypeStruct((N, D), jnp.float32),
        jax.ShapeDtypeStruct((N, D), jnp.float32),
    ],
)


@jax.jit
def kernel(features, edge_index, lin):
    src = edge_index[0]
    dst = edge_index[1]
    pad = EPAD - E
    srcp = jnp.concatenate([src, jnp.zeros((pad,), jnp.int32)])
    dstp = jnp.concatenate([dst, jnp.full((pad,), N, jnp.int32)])
    eidx = (srcp | (dstp << 16)).reshape(NW, EPT)
    featp = jnp.concatenate(
        [features, jnp.zeros((NPAD - N, D), jnp.float32)])
    znodes = jnp.zeros((NPAD, D), jnp.float32)

    deg_parts = _deg(eidx)
    dis, y1 = _prep(deg_parts, featp)
    p1 = _edge(y1, znodes, eidx)
    x1, y2 = _comb(p1, dis)
    p2 = _edge(y2, znodes, eidx)
    logp, out, x2 = _final(p2, dis, lin)
    return (logp, out, x1, x2)
